# baseline (speedup 1.0000x reference)
import jax
import jax.numpy as jnp
from jax.experimental import pallas as pl
from jax.experimental.pallas import tpu as pltpu

N = 2048
E = 65536
HEADS = 8
NHID = 8
GF = 15
NEG = 0.2


def _k1_body(wc_ref, btot_ref, x_ref, s_ref, o_ref):
    s = s_ref[...]
    for j in range(8):
        xb = x_ref[:, :, j, :]
        h = (wc_ref[0] * xb[:, 0, :] + wc_ref[1] * xb[:, 1, :]
             + wc_ref[2] * xb[:, 2, :] + wc_ref[3] * xb[:, 3, :])
        o_ref[:, j, :] = jnp.dot(h, s, preferred_element_type=jnp.float32) + btot_ref[0]


def _k1(x, cj_w, cj_b, cj2_w, cj2_b):
    wc = cj_w[0, :, 0]
    w2 = cj2_w[0, 0]
    btot = (cj_b[0] * jnp.sum(w2) + cj2_b[0]).reshape(1)
    LB = 1875
    L1 = 125
    BN = 64
    S = jnp.where(
        (jnp.arange(LB)[:, None] // 15) == jnp.arange(L1)[None, :],
        w2[jnp.arange(LB) % 15][:, None], 0.0).astype(jnp.float32)
    xr = x.reshape(N, 4, 8, LB)
    out = pl.pallas_call(
        _k1_body,
        grid=(N // BN,),
        in_specs=[
            pl.BlockSpec(memory_space=pltpu.SMEM),
            pl.BlockSpec(memory_space=pltpu.SMEM),
            pl.BlockSpec((BN, 4, 8, LB), lambda i: (i, 0, 0, 0)),
            pl.BlockSpec((LB, L1), lambda i: (0, 0)),
        ],
        out_specs=pl.BlockSpec((BN, 8, L1), lambda i: (i, 0, 0)),
        out_shape=jax.ShapeDtypeStruct((N, 8, L1), jnp.float32),
    )(wc, btot, xr, S)
    return out.reshape(N, 1000)


def _leaky(x):
    return jnp.where(x >= 0, x, NEG * x)


def _seg_softmax_apply(logits, dst, n):
    amax = jax.ops.segment_max(logits, dst, num_segments=n)
    e = jnp.exp(logits - amax[jnp.clip(dst, 0, n - 1)])
    e = jnp.where(dst[:, None] < n, e, 0.0)
    s = jax.ops.segment_sum(e, dst, num_segments=n)
    return e, s


def _gca_jnp(xl, gf, src, dst, att, att2, bias, bias2, heads, C, G):
    n = xl.shape[0]
    a1n = jnp.einsum('nhc,hc->nh', xl.reshape(n, heads, C), att[0])
    a2n = jnp.einsum('ng,hg->nh', gf, att2[0])
    l1 = _leaky(a1n[src] + a1n[dst])
    l2 = _leaky(a2n[src] + a2n[dst])
    e1, s1 = _seg_softmax_apply(l1, dst, n)
    e2, s2 = _seg_softmax_apply(l2, dst, n)
    xlh = xl.reshape(n, heads, C)
    U2 = jax.ops.segment_sum(e2[:, :, None] * xlh[src], dst, num_segments=n)
    U1 = jax.ops.segment_sum(e1[:, :, None] * gf[src][:, None, :], dst, num_segments=n)
    out = (U2 / (s2[:, :, None] + 1e-16)).reshape(n, heads * C) + bias
    out2 = jnp.mean(U1 / (s1[:, :, None] + 1e-16), axis=1) + bias2
    return out, out2


def kernel(x, edge_index, geneflow, cj_w, cj_b, cj2_w, cj2_b, fc1_w, fc1_b,
           fc2_w, fc2_b, c1_W, c1_b, c1_att, c1_att2, c1_bias, c1_bias2,
           c2_W, c2_b, c2_att, c2_att2, c2_bias, c2_bias2):
    n = N
    src0, dst0 = edge_index[0], edge_index[1]
    m = src0 != dst0
    dst0 = jnp.where(m, dst0, n)
    ar = jnp.arange(n, dtype=src0.dtype)
    src = jnp.concatenate([src0, ar])
    dst = jnp.concatenate([dst0, ar])

    h0 = _k1(x, cj_w, cj_b, cj2_w, cj2_b)

    xl = h0 @ c1_W.T + c1_b
    gf = geneflow @ fc1_w.T + fc1_b
    h, gf = _gca_jnp(xl, gf, src, dst, c1_att, c1_att2, c1_bias, c1_bias2,
                     HEADS, NHID, GF)
    h = jax.nn.relu(h)
    gf = jax.nn.relu(gf)
    gf = gf @ fc2_w.T + fc2_b
    xl2 = h @ c2_W.T + c2_b
    h2, gf2 = _gca_jnp(xl2, gf, src, dst, c2_att, c2_att2, c2_bias, c2_bias2,
                       1, 1, 1)
    return jax.nn.sigmoid(gf2.reshape(-1) + h2.reshape(-1))
